# baseline (speedup 1.0000x reference)
import jax
import jax.numpy as jnp
from jax.experimental import pallas as pl
from jax.experimental.pallas import tpu as pltpu

N_BLOCK = 200
G = 8


def _gat_block_kernel(nf_ref, nb_ref, asrc_ref, adst_ref, wc_ref, out_ref):
    nf = nf_ref[...]
    nb = nb_ref[...]
    asrc_v = asrc_ref[...]
    adst_v = adst_ref[...]
    wc = wc_ref[...]
    b, k, d = nb.shape

    e_src = jnp.sum(nf * asrc_v, axis=1, keepdims=True)
    nb3 = nb.reshape(b // G, G * k, d)
    adst_bc = jnp.broadcast_to(adst_v, (b // G, G, d))
    e3 = jax.lax.dot_general(
        adst_bc, nb3, (((2,), (2,)), ((0,), (0,))),
        preferred_element_type=jnp.float32)
    e2 = e3.reshape(b, G * k)
    lane0 = jax.lax.broadcasted_iota(jnp.int32, (b, G * k), 1) // k
    row0 = jax.lax.broadcasted_iota(jnp.int32, (b, G * k), 0) % G
    e_sel = jnp.where(lane0 == row0, e2, 0.0)
    e_dst = jnp.sum(e_sel.reshape(b, G, k), axis=1)
    e = e_src + e_dst
    e = jnp.where(e >= 0, e, 0.2 * e)
    e_max = jnp.max(e, axis=1, keepdims=True)
    ex = jnp.exp(e - e_max)
    alpha = ex / jnp.sum(ex, axis=1, keepdims=True)

    tiled = jnp.tile(alpha, (1, G))
    lane = jax.lax.broadcasted_iota(jnp.int32, (b, G * k), 1) // k
    row = jax.lax.broadcasted_iota(jnp.int32, (b, G * k), 0) % G
    adiag = jnp.where(lane == row, tiled, 0.0)

    a3 = adiag.reshape(b // G, G, G * k)
    nb3 = nb.reshape(b // G, G * k, d)
    xagg = jax.lax.dot_general(
        a3, nb3, (((2,), (1,)), ((0,), (0,))),
        preferred_element_type=jnp.float32).reshape(b, d)

    out_ref[...] = jnp.dot(xagg, wc, preferred_element_type=jnp.float32)


def kernel(node_feature, neighbor_nodes_feature, W, a_src, a_dst, weight):
    n, d = node_feature.shape
    k = neighbor_nodes_feature.shape[1]
    c = weight.shape[1]
    asrc_v = (W @ a_src[0]).reshape(1, d)
    adst_v = (W @ a_dst[0]).reshape(1, d)
    wc = W @ weight

    b = N_BLOCK
    grid = (n // b,)
    out = pl.pallas_call(
        _gat_block_kernel,
        grid=grid,
        in_specs=[
            pl.BlockSpec((b, d), lambda i: (i, 0)),
            pl.BlockSpec((b, k, d), lambda i: (i, 0, 0)),
            pl.BlockSpec((1, d), lambda i: (0, 0)),
            pl.BlockSpec((1, d), lambda i: (0, 0)),
            pl.BlockSpec((d, c), lambda i: (0, 0)),
        ],
        out_specs=pl.BlockSpec((b, c), lambda i: (i, 0)),
        out_shape=jax.ShapeDtypeStruct((n, c), jnp.float32),
        compiler_params=pltpu.CompilerParams(
            dimension_semantics=("arbitrary",),
        ),
    )(node_feature, neighbor_nodes_feature, asrc_v, adst_v, wc)
    return out


# speedup vs baseline: 1.3750x; 1.3750x over previous
import jax
import jax.numpy as jnp
from jax.experimental import pallas as pl
from jax.experimental.pallas import tpu as pltpu

N_BLOCK = 1000
G = 8


def _gat_block_kernel(nf_ref, nb_ref, asrc_ref, adst_ref, wc_ref, out_ref):
    nf = nf_ref[...]
    nb = nb_ref[...]
    asrc_v = asrc_ref[...]
    adst_v = adst_ref[...]
    wc = wc_ref[...]
    b, k, d = nb.shape

    e_src = jnp.sum(nf * asrc_v, axis=1, keepdims=True)
    nb3 = nb.reshape(b // G, G * k, d)
    adst_bc = jnp.broadcast_to(adst_v, (b // G, G, d))
    e3 = jax.lax.dot_general(
        adst_bc, nb3, (((2,), (2,)), ((0,), (0,))),
        preferred_element_type=jnp.float32)
    e2 = e3.reshape(b, G * k)
    lane0 = jax.lax.broadcasted_iota(jnp.int32, (b, G * k), 1) // k
    row0 = jax.lax.broadcasted_iota(jnp.int32, (b, G * k), 0) % G
    e_sel = jnp.where(lane0 == row0, e2, 0.0)
    e_dst = jnp.sum(e_sel.reshape(b, G, k), axis=1)
    e = e_src + e_dst
    e = jnp.where(e >= 0, e, 0.2 * e)
    e_max = jnp.max(e, axis=1, keepdims=True)
    ex = jnp.exp(e - e_max)
    alpha = ex / jnp.sum(ex, axis=1, keepdims=True)

    tiled = jnp.tile(alpha, (1, G))
    lane = jax.lax.broadcasted_iota(jnp.int32, (b, G * k), 1) // k
    row = jax.lax.broadcasted_iota(jnp.int32, (b, G * k), 0) % G
    adiag = jnp.where(lane == row, tiled, 0.0)

    a3 = adiag.reshape(b // G, G, G * k)
    nb3 = nb.reshape(b // G, G * k, d)
    xagg = jax.lax.dot_general(
        a3, nb3, (((2,), (1,)), ((0,), (0,))),
        preferred_element_type=jnp.float32).reshape(b, d)

    out_ref[...] = jnp.dot(xagg, wc, preferred_element_type=jnp.float32)


def kernel(node_feature, neighbor_nodes_feature, W, a_src, a_dst, weight):
    n, d = node_feature.shape
    k = neighbor_nodes_feature.shape[1]
    c = weight.shape[1]
    asrc_v = (W @ a_src[0]).reshape(1, d)
    adst_v = (W @ a_dst[0]).reshape(1, d)
    wc = W @ weight

    b = N_BLOCK
    grid = (n // b,)
    out = pl.pallas_call(
        _gat_block_kernel,
        grid=grid,
        in_specs=[
            pl.BlockSpec((b, d), lambda i: (i, 0)),
            pl.BlockSpec((b, k, d), lambda i: (i, 0, 0)),
            pl.BlockSpec((1, d), lambda i: (0, 0)),
            pl.BlockSpec((1, d), lambda i: (0, 0)),
            pl.BlockSpec((d, c), lambda i: (0, 0)),
        ],
        out_specs=pl.BlockSpec((b, c), lambda i: (i, 0)),
        out_shape=jax.ShapeDtypeStruct((n, c), jnp.float32),
        compiler_params=pltpu.CompilerParams(
            dimension_semantics=("arbitrary",),
        ),
    )(node_feature, neighbor_nodes_feature, asrc_v, adst_v, wc)
    return out


# speedup vs baseline: 1.3763x; 1.0010x over previous
import jax
import jax.numpy as jnp
from jax.experimental import pallas as pl
from jax.experimental.pallas import tpu as pltpu

N_BLOCK = 1000
G = 8


def _gat_block_kernel(nf_ref, nb_ref, asrc_ref, adst_ref, wc_ref, out_ref):
    nf = nf_ref[...]
    nb = nb_ref[...]
    asrc_v = asrc_ref[...]
    adst_v = adst_ref[...]
    wc = wc_ref[...]
    b, k, d = nb.shape

    e_src = jnp.sum(nf * asrc_v, axis=1, keepdims=True)
    nb3 = nb.reshape(b // G, G * k, d)
    adst_bc = jnp.broadcast_to(adst_v, (b // G, G, d))
    e3 = jax.lax.dot_general(
        adst_bc, nb3, (((2,), (2,)), ((0,), (0,))),
        preferred_element_type=jnp.float32)
    e2 = e3.reshape(b, G * k)
    lane0 = jax.lax.broadcasted_iota(jnp.int32, (b, G * k), 1) // k
    row0 = jax.lax.broadcasted_iota(jnp.int32, (b, G * k), 0) % G
    e_sel = jnp.where(lane0 == row0, e2, 0.0)
    e_dst = jnp.sum(e_sel.reshape(b, G, k), axis=1)
    e = e_src + e_dst
    e = jnp.where(e >= 0, e, 0.2 * e)
    e_max = jnp.max(e, axis=1, keepdims=True)
    ex = jnp.exp(e - e_max)
    alpha = ex / jnp.sum(ex, axis=1, keepdims=True)

    tiled = jnp.tile(alpha, (1, G))
    lane = jax.lax.broadcasted_iota(jnp.int32, (b, G * k), 1) // k
    row = jax.lax.broadcasted_iota(jnp.int32, (b, G * k), 0) % G
    adiag = jnp.where(lane == row, tiled, 0.0)

    a3 = adiag.reshape(b // G, G, G * k)
    nb3 = nb.reshape(b // G, G * k, d)
    xagg = jax.lax.dot_general(
        a3, nb3, (((2,), (1,)), ((0,), (0,))),
        preferred_element_type=jnp.float32).reshape(b, d)

    out_ref[...] = jnp.dot(xagg, wc, preferred_element_type=jnp.float32)


def kernel(node_feature, neighbor_nodes_feature, W, a_src, a_dst, weight):
    n, d = node_feature.shape
    k = neighbor_nodes_feature.shape[1]
    c = weight.shape[1]
    asrc_v = (W @ a_src[0]).reshape(1, d)
    adst_v = (W @ a_dst[0]).reshape(1, d)
    wc = W @ weight

    b = N_BLOCK
    grid = (n // b,)
    out = pl.pallas_call(
        _gat_block_kernel,
        grid=grid,
        in_specs=[
            pl.BlockSpec((b, d), lambda i: (i, 0)),
            pl.BlockSpec((b, k, d), lambda i: (i, 0, 0)),
            pl.BlockSpec((1, d), lambda i: (0, 0)),
            pl.BlockSpec((1, d), lambda i: (0, 0)),
            pl.BlockSpec((d, c), lambda i: (0, 0)),
        ],
        out_specs=pl.BlockSpec((b, c), lambda i: (i, 0)),
        out_shape=jax.ShapeDtypeStruct((n, c), jnp.float32),
        compiler_params=pltpu.CompilerParams(
            dimension_semantics=("parallel",),
        ),
    )(node_feature, neighbor_nodes_feature, asrc_v, adst_v, wc)
    return out
